# baseline (speedup 1.0000x reference)
import functools

import jax
import jax.numpy as jnp
from jax import lax
from jax.experimental import pallas as pl
from jax.experimental.pallas import tpu as pltpu
from jax.experimental.pallas import tpu_sc as plsc

_H = 64
_SEQ = 50
_NW = 32
_BT = 128
_L = 16
_NBUF = 5
_LOOK = 3


def _emb_body(ids_ref, tab_ref, pos_ref, out_ref, idx_v, pos_v, gbuf, tbuf,
              gsem, osem):
    wid = lax.axis_index("s") * 2 + lax.axis_index("c")

    pltpu.sync_copy(ids_ref.at[wid], idx_v)
    pltpu.sync_copy(pos_ref, pos_v)

    iota = jax.lax.iota(jnp.int32, _L)

    def start_gather(c, b):
        pltpu.async_copy(tab_ref.at[idx_v.at[c]], gbuf.at[b], gsem.at[b])

    def wait_gather(c, b):
        pltpu.make_async_copy(tab_ref.at[idx_v.at[c]], gbuf.at[b],
                              gsem.at[b]).wait()

    def wait_write(b):
        for tt in range(_H // 8):
            pltpu.make_async_copy(tbuf.at[b, tt], out_ref.at[0, tt, 0],
                                  osem.at[b]).wait()

    for b in range(_LOOK):
        start_gather(b, b)

    def outer(g, carry):
        for b in range(_NBUF):
            c = g * _NBUF + b
            wait_gather(c, b)

            @pl.when(c >= _NBUF)
            def _():
                wait_write(b)

            def hbody(h, carry2):
                col = jnp.full((_L,), h, jnp.int32)
                rowc = jnp.full((_L,), c, jnp.int32)
                pvec = plsc.load_gather(pos_v, [rowc, col])
                t8 = h // 8
                u = h % 8
                for gg in range(_BT // _L):
                    row = iota + (gg * _L)
                    vec = plsc.load_gather(gbuf.at[b], [row, col])
                    tbuf[b, t8, u, pl.ds(gg * _L, _L)] = vec + pvec
                return carry2

            lax.fori_loop(0, _H, hbody, 0)
            for tt in range(_H // 8):
                pltpu.async_copy(tbuf.at[b, tt], out_ref.at[c, tt, wid],
                                 osem.at[b])

            @pl.when(c + _LOOK < _SEQ)
            def _():
                start_gather(c + _LOOK, (b + _LOOK) % _NBUF)
        return carry

    lax.fori_loop(0, _SEQ // _NBUF, outer, 0)
    for b in range(_NBUF):
        wait_write(b)


def kernel(input_ids, token_table, pos_table):
    bsz, s = input_ids.shape
    h = token_table.shape[1]
    ids = jnp.transpose(
        input_ids.astype(jnp.int32).reshape(_NW, _BT, s), (0, 2, 1))
    mesh = plsc.VectorSubcoreMesh(core_axis_name="c", subcore_axis_name="s")
    run = functools.partial(
        pl.kernel,
        mesh=mesh,
        compiler_params=pltpu.CompilerParams(use_tc_tiling_on_sc=False, needs_layout_passes=False),
        out_type=jax.ShapeDtypeStruct((s, h // 8, _NW, 8, _BT), jnp.float32),
        scratch_types=[
            pltpu.VMEM((s, _BT), jnp.int32),
            pltpu.VMEM((64, h), jnp.float32),
            pltpu.VMEM((_NBUF, _BT, h), jnp.float32),
            pltpu.VMEM((_NBUF, h // 8, 8, _BT), jnp.float32),
            pltpu.SemaphoreType.DMA((_NBUF,)),
            pltpu.SemaphoreType.DMA((_NBUF,)),
        ],
    )(_emb_body)
    out5 = run(ids, token_table, pos_table)
    return jnp.transpose(out5, (2, 4, 0, 1, 3)).reshape(bsz, s, h)
